# baseline (speedup 1.0000x reference)
import functools

import jax
import jax.numpy as jnp
from jax.experimental import pallas as pl
from jax.experimental.pallas import tpu as pltpu
from jax.experimental.pallas import tpu_sc as plsc

B, S, T = 64, 200, 200
E, EC, H, NC = 128, 16, 128, 10
VC = 1000
G = 3 * H


def _sc_gather(table, idx_flat, value_dim, window):
    n = idx_flat.shape[0]
    idx2 = idx_flat.reshape(1, n).astype(jnp.int32)
    mesh = plsc.VectorSubcoreMesh(core_axis_name="core", subcore_axis_name="subcore")

    @pl.kernel(
        out_type=jax.ShapeDtypeStruct((n, value_dim), table.dtype),
        mesh=mesh,
    )
    def gather_kernel(tab_hbm, i_hbm, o_hbm):
        def body(i_vmem, o_vmem):
            pltpu.sync_copy(tab_hbm.at[i_vmem.at[0]], o_vmem)

        pltpu.emit_pipeline(
            body,
            grid=(n // window,),
            in_specs=[pl.BlockSpec((1, window), index_map=lambda i: (0, i))],
            out_specs=[pl.BlockSpec((window, value_dim), index_map=lambda i: (i, 0))],
            core_axis_name="subcore",
            dimension_semantics=(pltpu.PARALLEL,),
        )(i_hbm, o_hbm)

    return gather_kernel(table, idx2)


def _tc_body(
    src_e, trg_e, cn_ids, emb_cn,
    Wf0, Wfc, bf, Uf,
    Wb0, Wbc, bb, Ub,
    Wbrf, Wbrb, bbr,
    Wkf, Wkb, vrep,
    Wcf, Wcb, bc,
    WqUd, WdpE, WdpF, WdpB, Wph,
    Wdc, bd, Wpc, bp,
    dec_states, h_last, pre_outputs, clf,
    hs_f, hs_b, pk,
):
    f32 = jnp.float32
    dot = functools.partial(jnp.dot, preferred_element_type=f32)

    onehot = (jax.lax.broadcasted_iota(jnp.int32, (B, VC), 1)
              == cn_ids[...]).astype(f32)
    cn = dot(onehot, emb_cn[...])
    cnf = dot(cn, Wfc[...]) + bf[...]
    cnb = dot(cn, Wbc[...]) + bb[...]
    cnd = dot(cn, Wdc[...]) + bd[...]
    cnp = dot(cn, Wpc[...]) + bp[...]

    def gru(gx, gh, h):
        xr = gx[:, 0:H]
        xz = gx[:, H:2 * H]
        xn = gx[:, 2 * H:3 * H]
        hr = gh[:, 0:H]
        hz = gh[:, H:2 * H]
        hn = gh[:, 2 * H:3 * H]
        r = jax.nn.sigmoid(xr + hr)
        z = jax.nn.sigmoid(xz + hz)
        n = jnp.tanh(xn + r * hn)
        return (1.0 - z) * n + z * h

    bf16 = jnp.bfloat16

    def fstep(s, h):
        x = src_e[s]
        gx = dot(x, Wf0[...]) + cnf
        gh = dot(h, Uf[...])
        h = gru(gx, gh, h)
        hs_f[s] = h.astype(bf16)
        return h

    hf_last = jax.lax.fori_loop(0, S, fstep, jnp.zeros((B, H), f32))

    def bstep(s, h):
        i = S - 1 - s
        x = src_e[i]
        gx = dot(x, Wb0[...]) + cnb
        gh = dot(h, Ub[...])
        h = gru(gx, gh, h)
        hs_b[i] = h.astype(bf16)
        return h

    hb_last = jax.lax.fori_loop(0, S, bstep, jnp.zeros((B, H), f32))

    mf = jnp.mean(hs_f[...].astype(f32), axis=0)
    mb = jnp.mean(hs_b[...].astype(f32), axis=0)
    clf[...] = dot(mf, Wcf[...]) + dot(mb, Wcb[...]) + bc[...]

    hsf_flat = hs_f[...].reshape(S * B, H)
    hsb_flat = hs_b[...].reshape(S * B, H)
    pk[...] = (dot(hsf_flat, Wkf[...])
               + dot(hsb_flat, Wkb[...])).astype(bf16).reshape(S, B, H)

    h_dec0 = jnp.tanh(dot(hf_last, Wbrf[...]) + dot(hb_last, Wbrb[...]) + bbr[...])

    mhat = jnp.sum(jnp.abs(vrep[...].astype(f32)), axis=0,
                   keepdims=True).astype(bf16)

    def dstep(s, h):
        hq = dot(h, WqUd[...])
        q = hq[:, :H].astype(bf16)
        gh = hq[:, H:]
        t = jnp.tanh(pk[...] + q[None, :, :])
        scr = dot(t.reshape(S * B, H), vrep[...]).astype(bf16).reshape(S, B, H)
        e = jnp.exp(scr - mhat[None])
        rden = 1.0 / jnp.sum(e.astype(f32), axis=0)
        ctx_f = jnp.sum((e * hs_f[...]).astype(f32), axis=0) * rden
        ctx_b = jnp.sum((e * hs_b[...]).astype(f32), axis=0) * rden
        emb_o = dot(trg_e[s], WdpE[...])
        cf_o = dot(ctx_f, WdpF[...])
        cb_o = dot(ctx_b, WdpB[...])
        gx = emb_o[:, :G] + cf_o[:, :G] + cb_o[:, :G] + cnd
        h_new = gru(gx, gh, h)
        pre = emb_o[:, G:] + cf_o[:, G:] + cb_o[:, G:] + dot(h_new, Wph[...]) + cnp
        dec_states[s] = h_new
        pre_outputs[s] = pre
        return h_new

    hl = jax.lax.fori_loop(0, T, dstep, h_dec0)
    h_last[...] = hl


def _make_tc_call(interpret=False):
    f32 = jnp.float32
    return pl.pallas_call(
        _tc_body,
        out_shape=[
            jax.ShapeDtypeStruct((T, B, H), f32),
            jax.ShapeDtypeStruct((B, H), f32),
            jax.ShapeDtypeStruct((T, B, H), f32),
            jax.ShapeDtypeStruct((B, NC), f32),
        ],
        scratch_shapes=[
            pltpu.VMEM((S, B, H), jnp.bfloat16),
            pltpu.VMEM((S, B, H), jnp.bfloat16),
            pltpu.VMEM((S, B, H), jnp.bfloat16),
        ],
        compiler_params=pltpu.CompilerParams(
            vmem_limit_bytes=100 * 1024 * 1024,
        ),
        interpret=interpret,
    )


_tc_call = _make_tc_call()


def _build_args(p, src_e, trg_e, cn_ids):
    Wf, Wbk, Wd = p["Wf"], p["Wbk"], p["Wd"]
    Wbr, Wk, Wc, Wp = p["Wbr"], p["Wk"], p["Wc"], p["Wp"]
    return (
        src_e, trg_e, cn_ids, p["emb_cn"],
        Wf[:E], Wf[E:], p["bf"].reshape(1, G), p["Uf"],
        Wbk[:E], Wbk[E:], p["bbk"].reshape(1, G), p["Ubk"],
        Wbr[:H], Wbr[H:], p["bbr"].reshape(1, H),
        Wk[:H].astype(jnp.bfloat16), Wk[H:].astype(jnp.bfloat16),
        jnp.tile(p["v"].reshape(H, 1), (1, 128)).astype(jnp.bfloat16),
        Wc[:H], Wc[H:], p["bc"].reshape(1, NC),
        jnp.concatenate([p["Wq"], p["Ud"]], axis=1),
        jnp.concatenate([Wd[:E], Wp[:E]], axis=1),
        jnp.concatenate([Wd[E + EC:E + EC + H], Wp[E + EC + H:E + EC + 2 * H]], axis=1),
        jnp.concatenate([Wd[E + EC + H:], Wp[E + EC + 2 * H:]], axis=1),
        Wp[E + EC:E + EC + H],
        Wd[E:E + EC], p["bd"].reshape(1, G), Wp[E:E + EC], p["bp"].reshape(1, H),
    )


def kernel(src, trg, src_mask, trg_mask, src_lengths, trg_lengths, cn, params):
    p = params
    src_e = _sc_gather(p["emb_src"], src.T.reshape(-1), E, 128).reshape(S, B, E)
    trg_e = _sc_gather(p["emb_trg"], trg.T.reshape(-1), E, 128).reshape(T, B, E)
    cn_ids = cn.reshape(B, 1).astype(jnp.int32)
    dec_t, h_last, pre_t, clf = _tc_call(*_build_args(p, src_e, trg_e, cn_ids))
    return (dec_t.transpose(1, 0, 2), h_last, pre_t.transpose(1, 0, 2), clf)


# speedup vs baseline: 1.1226x; 1.1226x over previous
import functools

import jax
import jax.numpy as jnp
from jax.experimental import pallas as pl
from jax.experimental.pallas import tpu as pltpu
from jax.experimental.pallas import tpu_sc as plsc

B, S, T = 64, 200, 200
E, EC, H, NC = 128, 16, 128, 10
VC = 1000
G = 3 * H


def _sc_gather(table, idx_flat, value_dim, window):
    n = idx_flat.shape[0]
    idx2 = idx_flat.reshape(1, n).astype(jnp.int32)
    mesh = plsc.VectorSubcoreMesh(core_axis_name="core", subcore_axis_name="subcore")

    @pl.kernel(
        out_type=jax.ShapeDtypeStruct((n, value_dim), table.dtype),
        mesh=mesh,
    )
    def gather_kernel(tab_hbm, i_hbm, o_hbm):
        def body(i_vmem, o_vmem):
            pltpu.sync_copy(tab_hbm.at[i_vmem.at[0]], o_vmem)

        pltpu.emit_pipeline(
            body,
            grid=(n // window,),
            in_specs=[pl.BlockSpec((1, window), index_map=lambda i: (0, i))],
            out_specs=[pl.BlockSpec((window, value_dim), index_map=lambda i: (i, 0))],
            core_axis_name="subcore",
            dimension_semantics=(pltpu.PARALLEL,),
        )(i_hbm, o_hbm)

    return gather_kernel(table, idx2)


def _tc_body(
    src_e, trg_e, cn_ids, emb_cn,
    Wf0, Wfc, bf, Uf,
    Wb0, Wbc, bb, Ub,
    Wbrf, Wbrb, bbr,
    Wkf, Wkb, vrep,
    Wcf, Wcb, bc,
    WqUd, WdpE, WdpF, WdpB, Wph,
    Wdc, bd, Wpc, bp,
    dec_states, h_last, pre_outputs, clf,
    hs_f, hs_b, pk,
):
    f32 = jnp.float32
    dot = functools.partial(jnp.dot, preferred_element_type=f32)

    onehot = (jax.lax.broadcasted_iota(jnp.int32, (B, VC), 1)
              == cn_ids[...]).astype(f32)
    cn = dot(onehot, emb_cn[...])
    cnf = dot(cn, Wfc[...]) + bf[...]
    cnb = dot(cn, Wbc[...]) + bb[...]
    cnd = dot(cn, Wdc[...]) + bd[...]
    cnp = dot(cn, Wpc[...]) + bp[...]

    def gru(gx, gh, h):
        xr = gx[:, 0:H]
        xz = gx[:, H:2 * H]
        xn = gx[:, 2 * H:3 * H]
        hr = gh[:, 0:H]
        hz = gh[:, H:2 * H]
        hn = gh[:, 2 * H:3 * H]
        r = jax.nn.sigmoid(xr + hr)
        z = jax.nn.sigmoid(xz + hz)
        n = jnp.tanh(xn + r * hn)
        return (1.0 - z) * n + z * h

    bf16 = jnp.bfloat16

    def fstep(s, h):
        x = src_e[s]
        gx = dot(x, Wf0[...]) + cnf
        gh = dot(h, Uf[...])
        h = gru(gx, gh, h)
        hs_f[s] = h
        return h

    hf_last = jax.lax.fori_loop(0, S, fstep, jnp.zeros((B, H), f32))

    def bstep(s, h):
        i = S - 1 - s
        x = src_e[i]
        gx = dot(x, Wb0[...]) + cnb
        gh = dot(h, Ub[...])
        h = gru(gx, gh, h)
        hs_b[i] = h
        return h

    hb_last = jax.lax.fori_loop(0, S, bstep, jnp.zeros((B, H), f32))

    mf = jnp.mean(hs_f[...], axis=0)
    mb = jnp.mean(hs_b[...], axis=0)
    clf[...] = dot(mf, Wcf[...]) + dot(mb, Wcb[...]) + bc[...]

    hsf_flat = hs_f[...].reshape(S * B, H)
    hsb_flat = hs_b[...].reshape(S * B, H)
    pk[...] = (dot(hsf_flat, Wkf[...])
               + dot(hsb_flat, Wkb[...])).astype(bf16).reshape(S, B, H)

    h_dec0 = jnp.tanh(dot(hf_last, Wbrf[...]) + dot(hb_last, Wbrb[...]) + bbr[...])

    mhat = jnp.sum(jnp.abs(vrep[...].astype(f32)), axis=0, keepdims=True)

    def dstep(s, h):
        hq = dot(h, WqUd[...])
        q = hq[:, :H].astype(bf16)
        gh = hq[:, H:]
        t = jnp.tanh(pk[...] + q[None, :, :])
        scr = dot(t.reshape(S * B, H), vrep[...]).reshape(S, B, H)
        e = jnp.exp(scr - mhat[None])
        rden = 1.0 / jnp.sum(e, axis=0)
        ctx_f = jnp.sum(e * hs_f[...], axis=0) * rden
        ctx_b = jnp.sum(e * hs_b[...], axis=0) * rden
        emb_o = dot(trg_e[s], WdpE[...])
        cf_o = dot(ctx_f, WdpF[...])
        cb_o = dot(ctx_b, WdpB[...])
        gx = emb_o[:, :G] + cf_o[:, :G] + cb_o[:, :G] + cnd
        h_new = gru(gx, gh, h)
        pre = emb_o[:, G:] + cf_o[:, G:] + cb_o[:, G:] + dot(h_new, Wph[...]) + cnp
        dec_states[s] = h_new
        pre_outputs[s] = pre
        return h_new

    hl = jax.lax.fori_loop(0, T, dstep, h_dec0)
    h_last[...] = hl


def _make_tc_call(interpret=False):
    f32 = jnp.float32
    return pl.pallas_call(
        _tc_body,
        out_shape=[
            jax.ShapeDtypeStruct((T, B, H), f32),
            jax.ShapeDtypeStruct((B, H), f32),
            jax.ShapeDtypeStruct((T, B, H), f32),
            jax.ShapeDtypeStruct((B, NC), f32),
        ],
        scratch_shapes=[
            pltpu.VMEM((S, B, H), f32),
            pltpu.VMEM((S, B, H), f32),
            pltpu.VMEM((S, B, H), jnp.bfloat16),
        ],
        compiler_params=pltpu.CompilerParams(
            vmem_limit_bytes=100 * 1024 * 1024,
        ),
        interpret=interpret,
    )


_tc_call = _make_tc_call()


def _build_args(p, src_e, trg_e, cn_ids):
    Wf, Wbk, Wd = p["Wf"], p["Wbk"], p["Wd"]
    Wbr, Wk, Wc, Wp = p["Wbr"], p["Wk"], p["Wc"], p["Wp"]
    return (
        src_e, trg_e, cn_ids, p["emb_cn"],
        Wf[:E], Wf[E:], p["bf"].reshape(1, G), p["Uf"],
        Wbk[:E], Wbk[E:], p["bbk"].reshape(1, G), p["Ubk"],
        Wbr[:H], Wbr[H:], p["bbr"].reshape(1, H),
        Wk[:H], Wk[H:],
        jnp.tile(p["v"].reshape(H, 1), (1, 128)).astype(jnp.bfloat16),
        Wc[:H], Wc[H:], p["bc"].reshape(1, NC),
        jnp.concatenate([p["Wq"], p["Ud"]], axis=1),
        jnp.concatenate([Wd[:E], Wp[:E]], axis=1),
        jnp.concatenate([Wd[E + EC:E + EC + H], Wp[E + EC + H:E + EC + 2 * H]], axis=1),
        jnp.concatenate([Wd[E + EC + H:], Wp[E + EC + 2 * H:]], axis=1),
        Wp[E + EC:E + EC + H],
        Wd[E:E + EC], p["bd"].reshape(1, G), Wp[E:E + EC], p["bp"].reshape(1, H),
    )


def kernel(src, trg, src_mask, trg_mask, src_lengths, trg_lengths, cn, params):
    p = params
    src_e = _sc_gather(p["emb_src"], src.T.reshape(-1), E, 128).reshape(S, B, E)
    trg_e = _sc_gather(p["emb_trg"], trg.T.reshape(-1), E, 128).reshape(T, B, E)
    cn_ids = cn.reshape(B, 1).astype(jnp.int32)
    dec_t, h_last, pre_t, clf = _tc_call(*_build_args(p, src_e, trg_e, cn_ids))
    return (dec_t.transpose(1, 0, 2), h_last, pre_t.transpose(1, 0, 2), clf)


# speedup vs baseline: 1.2510x; 1.1143x over previous
import functools

import jax
import jax.numpy as jnp
from jax.experimental import pallas as pl
from jax.experimental.pallas import tpu as pltpu
from jax.experimental.pallas import tpu_sc as plsc

B, S, T = 64, 200, 200
E, EC, H, NC = 128, 16, 128, 10
VC = 1000
G = 3 * H


def _sc_gather(table, idx_flat, value_dim, window):
    n = idx_flat.shape[0]
    idx2 = idx_flat.reshape(1, n).astype(jnp.int32)
    mesh = plsc.VectorSubcoreMesh(core_axis_name="core", subcore_axis_name="subcore")

    @pl.kernel(
        out_type=jax.ShapeDtypeStruct((n, value_dim), table.dtype),
        mesh=mesh,
    )
    def gather_kernel(tab_hbm, i_hbm, o_hbm):
        def body(i_vmem, o_vmem):
            pltpu.sync_copy(tab_hbm.at[i_vmem.at[0]], o_vmem)

        pltpu.emit_pipeline(
            body,
            grid=(n // window,),
            in_specs=[pl.BlockSpec((1, window), index_map=lambda i: (0, i))],
            out_specs=[pl.BlockSpec((window, value_dim), index_map=lambda i: (i, 0))],
            core_axis_name="subcore",
            dimension_semantics=(pltpu.PARALLEL,),
        )(i_hbm, o_hbm)

    return gather_kernel(table, idx2)


def _tc_body(
    src_e, trg_e, cn_ids, emb_cn,
    Wf0, Wfc, bf, Uf,
    Wb0, Wbc, bb, Ub,
    Wbrf, Wbrb, bbr,
    Wkf, Wkb, vrep,
    Wcf, Wcb, bc,
    WqUd, WdpE, WdpF, WdpB, Wph,
    Wdc, bd, Wpc, bp,
    dec_states, h_last, pre_outputs, clf,
    hs_f, hs_b, pk, gxe,
):
    f32 = jnp.float32
    dot = functools.partial(jnp.dot, preferred_element_type=f32)

    onehot = (jax.lax.broadcasted_iota(jnp.int32, (B, VC), 1)
              == cn_ids[...]).astype(f32)
    cn = dot(onehot, emb_cn[...])
    cnf = dot(cn, Wfc[...]) + bf[...]
    cnb = dot(cn, Wbc[...]) + bb[...]
    cnd = dot(cn, Wdc[...]) + bd[...]
    cnp = dot(cn, Wpc[...]) + bp[...]

    def gru(gx, gh, h):
        xr = gx[:, 0:H]
        xz = gx[:, H:2 * H]
        xn = gx[:, 2 * H:3 * H]
        hr = gh[:, 0:H]
        hz = gh[:, H:2 * H]
        hn = gh[:, 2 * H:3 * H]
        r = jax.nn.sigmoid(xr + hr)
        z = jax.nn.sigmoid(xz + hz)
        n = jnp.tanh(xn + r * hn)
        return (1.0 - z) * n + z * h

    bf16 = jnp.bfloat16
    BK = 25
    NBLK = S // BK

    def fblk(b, h):
        base = b * BK
        xb = src_e[pl.ds(base, BK)].reshape(BK * B, E)
        gxe[...] = dot(xb, Wf0[...]).reshape(BK, B, G) + cnf[None]

        def fstep(i, h):
            gh = dot(h, Uf[...])
            h = gru(gxe[i], gh, h)
            hs_f[base + i] = h
            return h

        return jax.lax.fori_loop(0, BK, fstep, h)

    hf_last = jax.lax.fori_loop(0, NBLK, fblk, jnp.zeros((B, H), f32))

    def bblk(bi, h):
        base = (NBLK - 1 - bi) * BK
        xb = src_e[pl.ds(base, BK)].reshape(BK * B, E)
        gxe[...] = dot(xb, Wb0[...]).reshape(BK, B, G) + cnb[None]

        def bstep(i, h):
            idx = BK - 1 - i
            gh = dot(h, Ub[...])
            h = gru(gxe[idx], gh, h)
            hs_b[base + idx] = h
            return h

        return jax.lax.fori_loop(0, BK, bstep, h)

    hb_last = jax.lax.fori_loop(0, NBLK, bblk, jnp.zeros((B, H), f32))

    mf = jnp.mean(hs_f[...], axis=0)
    mb = jnp.mean(hs_b[...], axis=0)
    clf[...] = dot(mf, Wcf[...]) + dot(mb, Wcb[...]) + bc[...]

    hsf_flat = hs_f[...].reshape(S * B, H)
    hsb_flat = hs_b[...].reshape(S * B, H)
    pk[...] = (dot(hsf_flat, Wkf[...]) + dot(hsb_flat, Wkb[...])).reshape(S, B, H)

    h_dec0 = jnp.tanh(dot(hf_last, Wbrf[...]) + dot(hb_last, Wbrb[...]) + bbr[...])

    mhat = jnp.sum(jnp.abs(vrep[...].astype(f32)), axis=0, keepdims=True)

    def dstep(s, h):
        hq = dot(h, WqUd[...])
        q = hq[:, :H]
        gh = hq[:, H:]
        t = jnp.tanh(pk[...] + q[None, :, :]).astype(bf16)
        scr = dot(t.reshape(S * B, H), vrep[...]).reshape(S, B, H)
        e = jnp.exp(scr - mhat[None])
        rden = 1.0 / jnp.sum(e, axis=0)
        ctx_f = jnp.sum(e * hs_f[...], axis=0) * rden
        ctx_b = jnp.sum(e * hs_b[...], axis=0) * rden
        emb_o = dot(trg_e[s], WdpE[...])
        cf_o = dot(ctx_f, WdpF[...])
        cb_o = dot(ctx_b, WdpB[...])
        gx = emb_o[:, :G] + cf_o[:, :G] + cb_o[:, :G] + cnd
        h_new = gru(gx, gh, h)
        pre = emb_o[:, G:] + cf_o[:, G:] + cb_o[:, G:] + dot(h_new, Wph[...]) + cnp
        dec_states[s] = h_new
        pre_outputs[s] = pre
        return h_new

    hl = jax.lax.fori_loop(0, T, dstep, h_dec0)
    h_last[...] = hl


def _make_tc_call(interpret=False):
    f32 = jnp.float32
    return pl.pallas_call(
        _tc_body,
        out_shape=[
            jax.ShapeDtypeStruct((T, B, H), f32),
            jax.ShapeDtypeStruct((B, H), f32),
            jax.ShapeDtypeStruct((T, B, H), f32),
            jax.ShapeDtypeStruct((B, NC), f32),
        ],
        scratch_shapes=[
            pltpu.VMEM((S, B, H), f32),
            pltpu.VMEM((S, B, H), f32),
            pltpu.VMEM((S, B, H), f32),
            pltpu.VMEM((25, B, G), f32),
        ],
        input_output_aliases={0: 0, 1: 2},
        compiler_params=pltpu.CompilerParams(
            vmem_limit_bytes=100 * 1024 * 1024,
        ),
        interpret=interpret,
    )


_tc_call = _make_tc_call()


def _build_args(p, src_e, trg_e, cn_ids):
    Wf, Wbk, Wd = p["Wf"], p["Wbk"], p["Wd"]
    Wbr, Wk, Wc, Wp = p["Wbr"], p["Wk"], p["Wc"], p["Wp"]
    return (
        src_e, trg_e, cn_ids, p["emb_cn"],
        Wf[:E], Wf[E:], p["bf"].reshape(1, G), p["Uf"],
        Wbk[:E], Wbk[E:], p["bbk"].reshape(1, G), p["Ubk"],
        Wbr[:H], Wbr[H:], p["bbr"].reshape(1, H),
        Wk[:H], Wk[H:],
        jnp.tile(p["v"].reshape(H, 1), (1, 128)).astype(jnp.bfloat16),
        Wc[:H], Wc[H:], p["bc"].reshape(1, NC),
        jnp.concatenate([p["Wq"], p["Ud"]], axis=1),
        jnp.concatenate([Wd[:E], Wp[:E]], axis=1),
        jnp.concatenate([Wd[E + EC:E + EC + H], Wp[E + EC + H:E + EC + 2 * H]], axis=1),
        jnp.concatenate([Wd[E + EC + H:], Wp[E + EC + 2 * H:]], axis=1),
        Wp[E + EC:E + EC + H],
        Wd[E:E + EC], p["bd"].reshape(1, G), Wp[E:E + EC], p["bp"].reshape(1, H),
    )


def kernel(src, trg, src_mask, trg_mask, src_lengths, trg_lengths, cn, params):
    p = params
    src_e = _sc_gather(p["emb_src"], src.T.reshape(-1), E, 128).reshape(S, B, E)
    trg_e = _sc_gather(p["emb_trg"], trg.T.reshape(-1), E, 128).reshape(T, B, E)
    cn_ids = cn.reshape(B, 1).astype(jnp.int32)
    dec_t, h_last, pre_t, clf = _tc_call(*_build_args(p, src_e, trg_e, cn_ids))
    return (dec_t.transpose(1, 0, 2), h_last, pre_t.transpose(1, 0, 2), clf)


# speedup vs baseline: 1.2943x; 1.0347x over previous
import functools

import jax
import jax.numpy as jnp
from jax.experimental import pallas as pl
from jax.experimental.pallas import tpu as pltpu
from jax.experimental.pallas import tpu_sc as plsc

B, S, T = 64, 200, 200
E, EC, H, NC = 128, 16, 128, 10
VC = 1000
G = 3 * H


def _sc_gather(table, idx_flat, value_dim, window):
    n = idx_flat.shape[0]
    idx2 = idx_flat.reshape(1, n).astype(jnp.int32)
    mesh = plsc.VectorSubcoreMesh(core_axis_name="core", subcore_axis_name="subcore")

    @pl.kernel(
        out_type=jax.ShapeDtypeStruct((n, value_dim), table.dtype),
        mesh=mesh,
    )
    def gather_kernel(tab_hbm, i_hbm, o_hbm):
        def body(i_vmem, o_vmem):
            pltpu.sync_copy(tab_hbm.at[i_vmem.at[0]], o_vmem)

        pltpu.emit_pipeline(
            body,
            grid=(n // window,),
            in_specs=[pl.BlockSpec((1, window), index_map=lambda i: (0, i))],
            out_specs=[pl.BlockSpec((window, value_dim), index_map=lambda i: (i, 0))],
            core_axis_name="subcore",
            dimension_semantics=(pltpu.PARALLEL,),
        )(i_hbm, o_hbm)

    return gather_kernel(table, idx2)


def _tc_body(
    src_e, trg_e, cn_ids, emb_cn,
    Wf0, Wfc, bf, Uf,
    Wb0, Wbc, bb, Ub,
    Wbrf, Wbrb, bbr,
    Wkf, Wkb, vrep,
    Wcf, Wcb, bc,
    WqUd, WdpE, WdpF, WdpB, Wph,
    Wdc, bd, Wpc, bp,
    dec_states, h_last, pre_outputs, clf,
    hs_f, hs_b, pk, gxf, gxb,
):
    f32 = jnp.float32
    dot = functools.partial(jnp.dot, preferred_element_type=f32)

    onehot = (jax.lax.broadcasted_iota(jnp.int32, (B, VC), 1)
              == cn_ids[...]).astype(f32)
    cn = dot(onehot, emb_cn[...])
    cnf = dot(cn, Wfc[...]) + bf[...]
    cnb = dot(cn, Wbc[...]) + bb[...]
    cnd = dot(cn, Wdc[...]) + bd[...]
    cnp = dot(cn, Wpc[...]) + bp[...]

    def gru(gx, gh, h):
        xr = gx[:, 0:H]
        xz = gx[:, H:2 * H]
        xn = gx[:, 2 * H:3 * H]
        hr = gh[:, 0:H]
        hz = gh[:, H:2 * H]
        hn = gh[:, 2 * H:3 * H]
        r = jax.nn.sigmoid(xr + hr)
        z = jax.nn.sigmoid(xz + hz)
        n = jnp.tanh(xn + r * hn)
        return (1.0 - z) * n + z * h

    bf16 = jnp.bfloat16
    BK = 25
    NBLK = S // BK

    def eblk(b, carry):
        hf, hb = carry
        base_f = b * BK
        base_b = (NBLK - 1 - b) * BK
        xf = src_e[pl.ds(base_f, BK)].reshape(BK * B, E)
        xb = src_e[pl.ds(base_b, BK)].reshape(BK * B, E)
        gxf[...] = dot(xf, Wf0[...]).reshape(BK, B, G) + cnf[None]
        gxb[...] = dot(xb, Wb0[...]).reshape(BK, B, G) + cnb[None]

        def estep(i, c):
            hf, hb = c
            idx = BK - 1 - i
            ghf = dot(hf, Uf[...])
            ghb = dot(hb, Ub[...])
            hf = gru(gxf[i], ghf, hf)
            hb = gru(gxb[idx], ghb, hb)
            hs_f[base_f + i] = hf
            hs_b[base_b + idx] = hb
            return hf, hb

        return jax.lax.fori_loop(0, BK, estep, (hf, hb))

    h0 = jnp.zeros((B, H), f32)
    hf_last, hb_last = jax.lax.fori_loop(0, NBLK, eblk, (h0, h0))

    mf = jnp.mean(hs_f[...], axis=0)
    mb = jnp.mean(hs_b[...], axis=0)
    clf[...] = dot(mf, Wcf[...]) + dot(mb, Wcb[...]) + bc[...]

    hsf_flat = hs_f[...].reshape(S * B, H)
    hsb_flat = hs_b[...].reshape(S * B, H)
    pk[...] = (dot(hsf_flat, Wkf[...]) + dot(hsb_flat, Wkb[...])).reshape(S, B, H)

    h_dec0 = jnp.tanh(dot(hf_last, Wbrf[...]) + dot(hb_last, Wbrb[...]) + bbr[...])

    mhat = jnp.sum(jnp.abs(vrep[...].astype(f32)), axis=0, keepdims=True)

    def dstep(s, h):
        hq = dot(h, WqUd[...])
        q = hq[:, :H]
        gh = hq[:, H:]
        t = jnp.tanh(pk[...] + q[None, :, :]).astype(bf16)
        scr = dot(t.reshape(S * B, H), vrep[...]).reshape(S, B, H)
        e = jnp.exp(scr - mhat[None])
        rden = 1.0 / jnp.sum(e, axis=0)
        ctx_f = jnp.sum(e * hs_f[...], axis=0) * rden
        ctx_b = jnp.sum(e * hs_b[...], axis=0) * rden
        emb_o = dot(trg_e[s], WdpE[...])
        cf_o = dot(ctx_f, WdpF[...])
        cb_o = dot(ctx_b, WdpB[...])
        gx = emb_o[:, :G] + cf_o[:, :G] + cb_o[:, :G] + cnd
        h_new = gru(gx, gh, h)
        pre = emb_o[:, G:] + cf_o[:, G:] + cb_o[:, G:] + dot(h_new, Wph[...]) + cnp
        dec_states[s] = h_new
        pre_outputs[s] = pre
        return h_new

    hl = jax.lax.fori_loop(0, T, dstep, h_dec0)
    h_last[...] = hl


def _make_tc_call(interpret=False):
    f32 = jnp.float32
    return pl.pallas_call(
        _tc_body,
        out_shape=[
            jax.ShapeDtypeStruct((T, B, H), f32),
            jax.ShapeDtypeStruct((B, H), f32),
            jax.ShapeDtypeStruct((T, B, H), f32),
            jax.ShapeDtypeStruct((B, NC), f32),
        ],
        scratch_shapes=[
            pltpu.VMEM((S, B, H), f32),
            pltpu.VMEM((S, B, H), f32),
            pltpu.VMEM((S, B, H), f32),
            pltpu.VMEM((25, B, G), f32),
            pltpu.VMEM((25, B, G), f32),
        ],
        input_output_aliases={0: 0, 1: 2},
        compiler_params=pltpu.CompilerParams(
            vmem_limit_bytes=100 * 1024 * 1024,
        ),
        interpret=interpret,
    )


_tc_call = _make_tc_call()


def _build_args(p, src_e, trg_e, cn_ids):
    Wf, Wbk, Wd = p["Wf"], p["Wbk"], p["Wd"]
    Wbr, Wk, Wc, Wp = p["Wbr"], p["Wk"], p["Wc"], p["Wp"]
    return (
        src_e, trg_e, cn_ids, p["emb_cn"],
        Wf[:E], Wf[E:], p["bf"].reshape(1, G), p["Uf"],
        Wbk[:E], Wbk[E:], p["bbk"].reshape(1, G), p["Ubk"],
        Wbr[:H], Wbr[H:], p["bbr"].reshape(1, H),
        Wk[:H], Wk[H:],
        jnp.tile(p["v"].reshape(H, 1), (1, 128)).astype(jnp.bfloat16),
        Wc[:H], Wc[H:], p["bc"].reshape(1, NC),
        jnp.concatenate([p["Wq"], p["Ud"]], axis=1),
        jnp.concatenate([Wd[:E], Wp[:E]], axis=1),
        jnp.concatenate([Wd[E + EC:E + EC + H], Wp[E + EC + H:E + EC + 2 * H]], axis=1),
        jnp.concatenate([Wd[E + EC + H:], Wp[E + EC + 2 * H:]], axis=1),
        Wp[E + EC:E + EC + H],
        Wd[E:E + EC], p["bd"].reshape(1, G), Wp[E:E + EC], p["bp"].reshape(1, H),
    )


def kernel(src, trg, src_mask, trg_mask, src_lengths, trg_lengths, cn, params):
    p = params
    src_e = _sc_gather(p["emb_src"], src.T.reshape(-1), E, 128).reshape(S, B, E)
    trg_e = _sc_gather(p["emb_trg"], trg.T.reshape(-1), E, 128).reshape(T, B, E)
    cn_ids = cn.reshape(B, 1).astype(jnp.int32)
    dec_t, h_last, pre_t, clf = _tc_call(*_build_args(p, src_e, trg_e, cn_ids))
    return (dec_t.transpose(1, 0, 2), h_last, pre_t.transpose(1, 0, 2), clf)


# speedup vs baseline: 1.3318x; 1.0289x over previous
import functools

import jax
import jax.numpy as jnp
from jax.experimental import pallas as pl
from jax.experimental.pallas import tpu as pltpu
from jax.experimental.pallas import tpu_sc as plsc

B, S, T = 64, 200, 200
E, EC, H, NC = 128, 16, 128, 10
VC = 1000
G = 3 * H


WINDOW = 128


def _sc_gather2(tab_a, idx_a, tab_b, idx_b):
    n = idx_a.shape[0]
    ia = idx_a.reshape(1, n).astype(jnp.int32)
    ib = idx_b.reshape(1, n).astype(jnp.int32)
    mesh = plsc.VectorSubcoreMesh(core_axis_name="core", subcore_axis_name="subcore")
    out = jax.ShapeDtypeStruct((n, tab_a.shape[1]), tab_a.dtype)

    @pl.kernel(out_type=[out, out], mesh=mesh)
    def gather_kernel(ta_hbm, ia_hbm, tb_hbm, ib_hbm, oa_hbm, ob_hbm):
        core = jax.lax.axis_index("core")

        def pipe(tab, ih, oh):
            def body(i_vmem, o_vmem):
                pltpu.sync_copy(tab.at[i_vmem.at[0]], o_vmem)

            pltpu.emit_pipeline(
                body,
                grid=(n // WINDOW,),
                in_specs=[pl.BlockSpec((1, WINDOW), index_map=lambda i: (0, i))],
                out_specs=[pl.BlockSpec((WINDOW, tab.shape[1]),
                                        index_map=lambda i: (i, 0))],
                core_axis_name="subcore",
                dimension_semantics=(pltpu.PARALLEL,),
            )(ih, oh)

        @pl.when(core == 0)
        def _():
            pipe(ta_hbm, ia_hbm, oa_hbm)

        @pl.when(core == 1)
        def _():
            pipe(tb_hbm, ib_hbm, ob_hbm)

    return gather_kernel(tab_a, ia, tab_b, ib)


def _tc_body(
    src_e, trg_e, cn_ids, emb_cn,
    Wf0, Wfc, bf, Uf,
    Wb0, Wbc, bb, Ub,
    Wbrf, Wbrb, bbr,
    Wkf, Wkb, vrep,
    Wcf, Wcb, bc,
    WqUd, WdpE, WdpF, WdpB, Wph,
    Wdc, bd, Wpc, bp,
    dec_states, h_last, pre_outputs, clf,
    hs_f, hs_b, pk, gxf, gxb,
):
    f32 = jnp.float32
    dot = functools.partial(jnp.dot, preferred_element_type=f32)

    onehot = (jax.lax.broadcasted_iota(jnp.int32, (B, VC), 1)
              == cn_ids[...]).astype(f32)
    cn = dot(onehot, emb_cn[...])
    cnf = dot(cn, Wfc[...]) + bf[...]
    cnb = dot(cn, Wbc[...]) + bb[...]
    cnd = dot(cn, Wdc[...]) + bd[...]
    cnp = dot(cn, Wpc[...]) + bp[...]

    def gru(gx, gh, h):
        xr = gx[:, 0:H]
        xz = gx[:, H:2 * H]
        xn = gx[:, 2 * H:3 * H]
        hr = gh[:, 0:H]
        hz = gh[:, H:2 * H]
        hn = gh[:, 2 * H:3 * H]
        r = jax.nn.sigmoid(xr + hr)
        z = jax.nn.sigmoid(xz + hz)
        n = jnp.tanh(xn + r * hn)
        return (1.0 - z) * n + z * h

    bf16 = jnp.bfloat16
    BK = 25
    NBLK = S // BK

    def eblk(b, carry):
        hf, hb = carry
        base_f = b * BK
        base_b = (NBLK - 1 - b) * BK
        xf = src_e[pl.ds(base_f, BK)].reshape(BK * B, E)
        xb = src_e[pl.ds(base_b, BK)].reshape(BK * B, E)
        gxf[...] = dot(xf, Wf0[...]).reshape(BK, B, G) + cnf[None]
        gxb[...] = dot(xb, Wb0[...]).reshape(BK, B, G) + cnb[None]

        def estep(i, c):
            hf, hb = c
            idx = BK - 1 - i
            ghf = dot(hf, Uf[...])
            ghb = dot(hb, Ub[...])
            hf = gru(gxf[i], ghf, hf)
            hb = gru(gxb[idx], ghb, hb)
            hs_f[base_f + i] = hf
            hs_b[base_b + idx] = hb
            return hf, hb

        return jax.lax.fori_loop(0, BK, estep, (hf, hb))

    h0 = jnp.zeros((B, H), f32)
    hf_last, hb_last = jax.lax.fori_loop(0, NBLK, eblk, (h0, h0))

    mf = jnp.mean(hs_f[...], axis=0)
    mb = jnp.mean(hs_b[...], axis=0)
    clf[...] = dot(mf, Wcf[...]) + dot(mb, Wcb[...]) + bc[...]

    hsf_flat = hs_f[...].reshape(S * B, H)
    hsb_flat = hs_b[...].reshape(S * B, H)
    pk[...] = (dot(hsf_flat, Wkf[...]) + dot(hsb_flat, Wkb[...])).reshape(S, B, H)

    h_dec0 = jnp.tanh(dot(hf_last, Wbrf[...]) + dot(hb_last, Wbrb[...]) + bbr[...])

    mhat = jnp.sum(jnp.abs(vrep[...].astype(f32)), axis=0, keepdims=True)

    def dstep(s, h):
        hq = dot(h, WqUd[...])
        q = hq[:, :H]
        gh = hq[:, H:]
        t = jnp.tanh(pk[...] + q[None, :, :]).astype(bf16)
        scr = dot(t.reshape(S * B, H), vrep[...]).reshape(S, B, H)
        e = jnp.exp(scr - mhat[None])
        rden = 1.0 / jnp.sum(e, axis=0)
        ctx_f = jnp.sum(e * hs_f[...], axis=0) * rden
        ctx_b = jnp.sum(e * hs_b[...], axis=0) * rden
        emb_o = dot(trg_e[s], WdpE[...])
        cf_o = dot(ctx_f, WdpF[...])
        cb_o = dot(ctx_b, WdpB[...])
        gx = emb_o[:, :G] + cf_o[:, :G] + cb_o[:, :G] + cnd
        h_new = gru(gx, gh, h)
        pre = emb_o[:, G:] + cf_o[:, G:] + cb_o[:, G:] + dot(h_new, Wph[...]) + cnp
        dec_states[s] = h_new
        pre_outputs[s] = pre
        return h_new

    hl = jax.lax.fori_loop(0, T, dstep, h_dec0)
    h_last[...] = hl


def _make_tc_call(interpret=False):
    f32 = jnp.float32
    return pl.pallas_call(
        _tc_body,
        out_shape=[
            jax.ShapeDtypeStruct((T, B, H), f32),
            jax.ShapeDtypeStruct((B, H), f32),
            jax.ShapeDtypeStruct((T, B, H), f32),
            jax.ShapeDtypeStruct((B, NC), f32),
        ],
        scratch_shapes=[
            pltpu.VMEM((S, B, H), f32),
            pltpu.VMEM((S, B, H), f32),
            pltpu.VMEM((S, B, H), f32),
            pltpu.VMEM((25, B, G), f32),
            pltpu.VMEM((25, B, G), f32),
        ],
        input_output_aliases={0: 0, 1: 2},
        compiler_params=pltpu.CompilerParams(
            vmem_limit_bytes=100 * 1024 * 1024,
        ),
        interpret=interpret,
    )


_tc_call = _make_tc_call()


def _build_args(p, src_e, trg_e, cn_ids):
    Wf, Wbk, Wd = p["Wf"], p["Wbk"], p["Wd"]
    Wbr, Wk, Wc, Wp = p["Wbr"], p["Wk"], p["Wc"], p["Wp"]
    return (
        src_e, trg_e, cn_ids, p["emb_cn"],
        Wf[:E], Wf[E:], p["bf"].reshape(1, G), p["Uf"],
        Wbk[:E], Wbk[E:], p["bbk"].reshape(1, G), p["Ubk"],
        Wbr[:H], Wbr[H:], p["bbr"].reshape(1, H),
        Wk[:H], Wk[H:],
        jnp.tile(p["v"].reshape(H, 1), (1, 128)).astype(jnp.bfloat16),
        Wc[:H], Wc[H:], p["bc"].reshape(1, NC),
        jnp.concatenate([p["Wq"], p["Ud"]], axis=1),
        jnp.concatenate([Wd[:E], Wp[:E]], axis=1),
        jnp.concatenate([Wd[E + EC:E + EC + H], Wp[E + EC + H:E + EC + 2 * H]], axis=1),
        jnp.concatenate([Wd[E + EC + H:], Wp[E + EC + 2 * H:]], axis=1),
        Wp[E + EC:E + EC + H],
        Wd[E:E + EC], p["bd"].reshape(1, G), Wp[E:E + EC], p["bp"].reshape(1, H),
    )


def kernel(src, trg, src_mask, trg_mask, src_lengths, trg_lengths, cn, params):
    p = params
    src_e, trg_e = _sc_gather2(p["emb_src"], src.T.reshape(-1),
                               p["emb_trg"], trg.T.reshape(-1))
    src_e = src_e.reshape(S, B, E)
    trg_e = trg_e.reshape(T, B, E)
    cn_ids = cn.reshape(B, 1).astype(jnp.int32)
    dec_t, h_last, pre_t, clf = _tc_call(*_build_args(p, src_e, trg_e, cn_ids))
    return (dec_t.transpose(1, 0, 2), h_last, pre_t.transpose(1, 0, 2), clf)
